# baseline (speedup 1.0000x reference)
import functools

import jax
import jax.numpy as jnp
from jax import lax
from jax.experimental import pallas as pl
from jax.experimental.pallas import tpu as pltpu
from jax.experimental.pallas import tpu_sc as plsc

VOCAB = 30522
MAX_POS = 2048
HIDDEN = 768
BATCH = 4
SEQ = 2048

NC = 2
NS = 16
NW = NC * NS
LANES = 16
ROWS = BATCH * SEQ
ROWS_PER_W = ROWS // NW
CHUNK = 64
NCHUNK = ROWS_PER_W // CHUNK
NVEC = HIDDEN // LANES


def _hsum16(x):
    dnums = lax.GatherDimensionNumbers(
        offset_dims=(), collapsed_slice_dims=(0,), start_index_map=(0,))
    for sh in (8, 4, 2, 1):
        idx = lax.iota(jnp.int32, LANES) ^ sh
        x = x + lax.gather(x, idx[:, None], dnums, (1,),
                           mode=lax.GatherScatterMode.PROMISE_IN_BOUNDS)
    return x


def _rsqrt16(v):
    bits = plsc.bitcast(v, jnp.int32)
    y = plsc.bitcast(jnp.int32(0x5F3759DF) - (bits >> 1), jnp.float32)
    half = v * 0.5
    for _ in range(3):
        y = y * (1.5 - half * y * y)
    return y


def _tec_body(ids_hbm, word_hbm, pos_hbm, gamma_hbm, beta_hbm, out_hbm,
              idx_v, word_v, pos_v, gamma_v, beta_v, sem):
    cid = lax.axis_index("c")
    sid = lax.axis_index("s")
    wid = sid * NC + cid
    wbase = wid * ROWS_PER_W

    pltpu.sync_copy(gamma_hbm, gamma_v)
    pltpu.sync_copy(beta_hbm, beta_v)

    for c in range(NCHUNK):
        base = wbase + c * CHUNK
        pos_base = lax.rem(base, MAX_POS)
        pltpu.sync_copy(ids_hbm.at[pl.ds(base, CHUNK)], idx_v)
        pltpu.async_copy(word_hbm.at[idx_v], word_v, sem).wait()
        pltpu.sync_copy(pos_hbm.at[pl.ds(pos_base, CHUNK)], pos_v)

        def row_body(r, _):
            acc = jnp.zeros((LANES,), jnp.float32)
            acc2 = jnp.zeros((LANES,), jnp.float32)
            for j in range(NVEC):
                sl = pl.ds(j * LANES, LANES)
                x = word_v[r, sl] + pos_v[r, sl]
                word_v[r, sl] = x
                acc = acc + x
                acc2 = acc2 + x * x
            mean_v = _hsum16(acc) * (1.0 / HIDDEN)
            var_v = _hsum16(acc2) * (1.0 / HIDDEN) - mean_v * mean_v
            rstd = _rsqrt16(var_v + 1e-12)
            for j in range(NVEC):
                sl = pl.ds(j * LANES, LANES)
                x = word_v[r, sl]
                word_v[r, sl] = (x - mean_v) * rstd * gamma_v[sl] + beta_v[sl]
            return _

        lax.fori_loop(0, CHUNK, row_body, None)
        pltpu.sync_copy(word_v, out_hbm.at[pl.ds(base, CHUNK)])


def kernel(input_ids, word_table, pos_table, gamma, beta):
    ids_flat = input_ids.reshape(-1).astype(jnp.int32)
    mesh = plsc.VectorSubcoreMesh(core_axis_name="c", subcore_axis_name="s")
    call = pl.kernel(
        _tec_body,
        mesh=mesh,
        out_type=jax.ShapeDtypeStruct((ROWS, HIDDEN), jnp.float32),
        scratch_types=[
            pltpu.VMEM((CHUNK,), jnp.int32),
            pltpu.VMEM((CHUNK, HIDDEN), jnp.float32),
            pltpu.VMEM((CHUNK, HIDDEN), jnp.float32),
            pltpu.VMEM((HIDDEN,), jnp.float32),
            pltpu.VMEM((HIDDEN,), jnp.float32),
            pltpu.SemaphoreType.DMA,
        ],
        compiler_params=pltpu.CompilerParams(needs_layout_passes=False),
    )
    out = call(ids_flat, word_table, pos_table, gamma, beta)
    return out.reshape(BATCH, SEQ, HIDDEN)
